# baseline (speedup 1.0000x reference)
import functools

import jax
import jax.numpy as jnp
from jax import lax
from jax.experimental import pallas as pl
from jax.experimental.pallas import tpu as pltpu
from jax.experimental.pallas import tpu_sc as plsc

_B, _L, _D = 64, 1024, 128
_NW = 32
_ROWS_PER_W = _B * _L // _NW
_C = 256
_NCHUNK = _ROWS_PER_W // _C
_LANES = 16


@functools.partial(jax.jit, donate_argnums=())
def _sc_embed(x_flat, token_table, pos_table):
  mesh = plsc.VectorSubcoreMesh(core_axis_name="c", subcore_axis_name="s")

  @functools.partial(
      pl.kernel,
      mesh=mesh,
      out_type=jax.ShapeDtypeStruct((_B * _L, _D), jnp.float32),
      scratch_types=[
          pltpu.VMEM((_C,), jnp.int32),
          pltpu.VMEM((_C, _D), jnp.float32),
          pltpu.VMEM((_C, _D), jnp.float32),
          pltpu.SemaphoreType.DMA,
      ],
  )
  def k(x_hbm, tok_hbm, pos_hbm, out_hbm, idx_v, rows_v, pos_v, sem):
    wid = lax.axis_index("s") * 2 + lax.axis_index("c")
    base = wid * _ROWS_PER_W
    for c in range(_NCHUNK):
      row0 = base + c * _C
      pos0 = (c * _C) % _L
      pltpu.sync_copy(x_hbm.at[pl.ds(row0, _C)], idx_v)
      gather = pltpu.async_copy(tok_hbm.at[idx_v], rows_v, sem)
      pltpu.sync_copy(pos_hbm.at[pl.ds(pos0, _C), :], pos_v)
      gather.wait()

      def add_body(r, _):
        for j in range(_D // _LANES):
          o = j * _LANES
          rows_v[r, pl.ds(o, _LANES)] = (
              rows_v[r, pl.ds(o, _LANES)] + pos_v[r, pl.ds(o, _LANES)]
          )
        return 0

      lax.fori_loop(0, _C, add_body, 0)
      pltpu.sync_copy(rows_v, out_hbm.at[pl.ds(row0, _C), :])

  return k(x_flat, token_table, pos_table)


def kernel(x, token_table, pos_table):
  out = _sc_embed(x.reshape(-1), token_table, pos_table)
  return out.reshape(_B, _L, _D)


# speedup vs baseline: 1.3691x; 1.3691x over previous
import functools

import jax
import jax.numpy as jnp
from jax import lax
from jax.experimental import pallas as pl
from jax.experimental.pallas import tpu as pltpu
from jax.experimental.pallas import tpu_sc as plsc

_B, _L, _D = 64, 1024, 128
_NW = 32
_ROWS_PER_W = _B * _L // _NW
_C = 128
_NU = _L // _C
_LANES = 16


@jax.jit
def _sc_embed(x2, token_table, pos_table):
  mesh = plsc.VectorSubcoreMesh(core_axis_name="c", subcore_axis_name="s")

  @functools.partial(
      pl.kernel,
      mesh=mesh,
      out_type=jax.ShapeDtypeStruct((_B * _L, _D), jnp.float32),
      scratch_types=[
          pltpu.VMEM((2 * _NU, _C), jnp.int32),
          pltpu.VMEM((2 * _C, _D), jnp.float32),
          pltpu.VMEM((2 * _C, _D), jnp.float32),
          pltpu.VMEM((_C, _D), jnp.float32),
          pltpu.VMEM((_C, _D), jnp.float32),
          pltpu.SemaphoreType.DMA,
          pltpu.SemaphoreType.DMA,
          pltpu.SemaphoreType.DMA,
          pltpu.SemaphoreType.DMA,
          pltpu.SemaphoreType.DMA,
          pltpu.SemaphoreType.DMA,
      ],
  )
  def k(x_hbm, tok_hbm, pos_hbm, out_hbm,
        idx_v, rows0, rows1, pos0, pos1, g0, g1, p0, p1, o0, o1):
    wid = lax.axis_index("s") * 2 + lax.axis_index("c")
    base = wid * _ROWS_PER_W
    pltpu.sync_copy(x_hbm.at[pl.ds(wid * 2 * _NU, 2 * _NU), :], idx_v)

    rows = [rows0, rows1]
    posb = [pos0, pos1]
    gsem = [g0, g1]
    psem = [p0, p1]
    osem = [o0, o1]

    def start(u):
      b = u % 2
      ga = pltpu.async_copy(
          tok_hbm.at[idx_v.at[u]], rows[b].at[pl.ds(0, _C)], gsem[b])
      gb = pltpu.async_copy(
          tok_hbm.at[idx_v.at[_NU + u]], rows[b].at[pl.ds(_C, _C)], gsem[b])
      pp = pltpu.async_copy(
          pos_hbm.at[pl.ds(u * _C, _C), :], posb[b], psem[b])
      return ga, gb, pp

    in_flight = {0: start(0)}
    out_flight = {}
    for u in range(_NU):
      b = u % 2
      if u + 1 < _NU:
        if u - 1 in out_flight:
          for h in out_flight.pop(u - 1):
            h.wait()
        in_flight[u + 1] = start(u + 1)
      ga, gb, pp = in_flight.pop(u)
      ga.wait()
      gb.wait()
      pp.wait()

      def add_body(r, _):
        for j in range(_D // _LANES):
          o = j * _LANES
          p = posb[b][r, pl.ds(o, _LANES)]
          rows[b][r, pl.ds(o, _LANES)] = rows[b][r, pl.ds(o, _LANES)] + p
          rows[b][_C + r, pl.ds(o, _LANES)] = (
              rows[b][_C + r, pl.ds(o, _LANES)] + p)
        return 0

      lax.fori_loop(0, _C, add_body, 0)

      oa = pltpu.async_copy(
          rows[b].at[pl.ds(0, _C)],
          out_hbm.at[pl.ds(base + u * _C, _C), :], osem[b])
      ob = pltpu.async_copy(
          rows[b].at[pl.ds(_C, _C)],
          out_hbm.at[pl.ds(base + _L + u * _C, _C), :], osem[b])
      out_flight[u] = (oa, ob)

    for u, hs in sorted(out_flight.items()):
      for h in hs:
        h.wait()

  return k(x2, token_table, pos_table)


def kernel(x, token_table, pos_table):
  out = _sc_embed(x.reshape(-1, _C), token_table, pos_table)
  return out.reshape(_B, _L, _D)


# speedup vs baseline: 1.7205x; 1.2567x over previous
import functools

import jax
import jax.numpy as jnp
from jax import lax
from jax.experimental import pallas as pl
from jax.experimental.pallas import tpu as pltpu
from jax.experimental.pallas import tpu_sc as plsc

_B, _L, _D = 64, 1024, 128
_SG = 8
_LG = 256
_V = 32
_NU = _LG // _V
_LANES = 16


@jax.jit
def _sc_embed(x, token_table, pos_table):
  mesh = plsc.VectorSubcoreMesh(core_axis_name="c", subcore_axis_name="s")

  @functools.partial(
      pl.kernel,
      mesh=mesh,
      out_type=jax.ShapeDtypeStruct((_B, _L, _D), jnp.float32),
      scratch_types=[
          pltpu.VMEM((_SG, _LG), jnp.int32),
          pltpu.VMEM((_LG, _D), jnp.float32),
          pltpu.VMEM((_SG, _V, _D), jnp.float32),
          pltpu.VMEM((_SG, _V, _D), jnp.float32),
          pltpu.SemaphoreType.DMA,
          pltpu.SemaphoreType.DMA,
          pltpu.SemaphoreType.DMA,
          pltpu.SemaphoreType.DMA,
          pltpu.SemaphoreType.DMA,
      ],
  )
  def k(x_hbm, tok_hbm, pos_hbm, out_hbm,
        idx_v, pos_v, rows0, rows1, psem, g0, g1, o0, o1):
    wid = lax.axis_index("s") * 2 + lax.axis_index("c")
    g0row = (wid // 4) * _SG
    l0 = (wid % 4) * _LG

    pltpu.sync_copy(x_hbm.at[pl.ds(g0row, _SG), pl.ds(l0, _LG)], idx_v)
    pos_h = pltpu.async_copy(pos_hbm.at[pl.ds(l0, _LG), :], pos_v, psem)

    rows = [rows0, rows1]
    gsem = [g0, g1]
    osem = [o0, o1]

    def start(v):
      b = v % 2
      return [
          pltpu.async_copy(
              tok_hbm.at[idx_v.at[s, pl.ds(v * _V, _V)]],
              rows[b].at[s], gsem[b])
          for s in range(_SG)
      ]

    in_flight = {0: start(0)}
    out_flight = {}
    for v in range(_NU):
      b = v % 2
      if v + 1 < _NU:
        if v - 1 in out_flight:
          for h in out_flight.pop(v - 1):
            h.wait()
        in_flight[v + 1] = start(v + 1)
      for h in in_flight.pop(v):
        h.wait()
      if v == 0:
        pos_h.wait()

      def add_body(i, _):
        for j in range(_D // _LANES):
          o = j * _LANES
          p = pos_v[v * _V + i, pl.ds(o, _LANES)]
          for s in range(_SG):
            rows[b][s, i, pl.ds(o, _LANES)] = (
                rows[b][s, i, pl.ds(o, _LANES)] + p)
        return 0

      lax.fori_loop(0, _V, add_body, 0)

      out_flight[v] = [
          pltpu.async_copy(
              rows[b].at[s],
              out_hbm.at[g0row + s, pl.ds(l0 + v * _V, _V), :], osem[b])
          for s in range(_SG)
      ]

    for v in sorted(out_flight):
      for h in out_flight[v]:
        h.wait()

  return k(x, token_table, pos_table)


def kernel(x, token_table, pos_table):
  return _sc_embed(x, token_table, pos_table)
